# Initial kernel scaffold; baseline (speedup 1.0000x reference)
import jax
import jax.numpy as jnp
from jax.experimental import pallas as pl


def kernel(token_ids, table):
    raise NotImplementedError("write your pallas kernel here")


# speedup vs baseline: 4.0780x; 4.0780x over previous
import functools

import jax
import jax.numpy as jnp
from jax import lax
from jax.experimental import pallas as pl
from jax.experimental.pallas import tpu as pltpu
from jax.experimental.pallas import tpu_sc as plsc

VOCAB = 100000
EMB = 64
BATCH = 4096
SEQ = 200

B = BATCH * SEQ
NC = 2
NS = 16
NW = NC * NS
BPW = B // NW
CH = 800
NCHUNK = BPW // CH

_mesh = plsc.VectorSubcoreMesh(core_axis_name="c", subcore_axis_name="s")


@functools.partial(
    pl.kernel,
    mesh=_mesh,
    out_type=jax.ShapeDtypeStruct((B, EMB), jnp.float32),
    compiler_params=pltpu.CompilerParams(use_tc_tiling_on_sc=False),
    scratch_types=[
        pltpu.VMEM((CH,), jnp.int32),
        pltpu.VMEM((CH, EMB), jnp.float32),
        pltpu.SemaphoreType.DMA,
    ],
)
def _gather_kernel(idx_hbm, table_hbm, out_hbm, idx_v, rows_v, sem):
    wid = lax.axis_index("s") * NC + lax.axis_index("c")
    base = wid * BPW

    def body(i, carry):
        start = base + i * CH
        pltpu.sync_copy(idx_hbm.at[pl.ds(start, CH)], idx_v)
        pltpu.async_copy(table_hbm.at[idx_v], rows_v, sem).wait()
        pltpu.sync_copy(rows_v, out_hbm.at[pl.ds(start, CH)])
        return carry

    lax.fori_loop(0, NCHUNK, body, 0)


def kernel(token_ids, table):
    idx = token_ids.reshape(B).astype(jnp.int32)
    out = _gather_kernel(idx, table)
    return out.reshape(BATCH, SEQ, EMB)


# speedup vs baseline: 4.2040x; 1.0309x over previous
import functools

import jax
import jax.numpy as jnp
from jax import lax
from jax.experimental import pallas as pl
from jax.experimental.pallas import tpu as pltpu
from jax.experimental.pallas import tpu_sc as plsc

VOCAB = 100000
EMB = 64
BATCH = 4096
SEQ = 200

B = BATCH * SEQ
NC = 2
NS = 16
NW = NC * NS
BPW = B // NW
CH = 800
NCHUNK = BPW // CH
NBUF = 2
NGROUPS = NCHUNK // NBUF

_mesh = plsc.VectorSubcoreMesh(core_axis_name="c", subcore_axis_name="s")


@functools.partial(
    pl.kernel,
    mesh=_mesh,
    out_type=jax.ShapeDtypeStruct((B, EMB), jnp.float32),
    compiler_params=pltpu.CompilerParams(use_tc_tiling_on_sc=False),
    scratch_types=[
        pltpu.VMEM((BPW,), jnp.int32),
        pltpu.VMEM((NBUF, CH, EMB), jnp.float32),
        pltpu.SemaphoreType.DMA,
        pltpu.SemaphoreType.DMA,
        pltpu.SemaphoreType.DMA,
        pltpu.SemaphoreType.DMA,
    ],
)
def _gather_kernel(idx_hbm, table_hbm, out_hbm, idx_v, rows_v, g0, g1, o0, o1):
    gsem = (g0, g1)
    osem = (o0, o1)
    wid = lax.axis_index("s") * NC + lax.axis_index("c")
    base = wid * BPW
    pltpu.sync_copy(idx_hbm.at[pl.ds(base, BPW)], idx_v)

    def gather_desc(i, b):
        return pltpu.make_async_copy(
            table_hbm.at[idx_v.at[pl.ds(i * CH, CH)]], rows_v.at[b], gsem[b]
        )

    def wb_desc(i, b):
        return pltpu.make_async_copy(
            rows_v.at[b], out_hbm.at[pl.ds(base + i * CH, CH)], osem[b]
        )

    for b in range(NBUF):
        gather_desc(b, b).start()

    def group(g, carry):
        i0 = g * NBUF
        for b in range(NBUF):
            gather_desc(i0 + b, b).wait()
            wb_desc(i0 + b, b).start()
        for b in range(NBUF):
            wb_desc(i0 + b, b).wait()
            gather_desc(i0 + NBUF + b, b).start()
        return carry

    lax.fori_loop(0, NGROUPS - 1, group, 0)

    i0 = (NGROUPS - 1) * NBUF
    for b in range(NBUF):
        gather_desc(i0 + b, b).wait()
        wb_desc(i0 + b, b).start()
    for b in range(NBUF):
        wb_desc(i0 + b, b).wait()


def kernel(token_ids, table):
    idx = token_ids.reshape(B).astype(jnp.int32)
    out = _gather_kernel(idx, table)
    return out.reshape(BATCH, SEQ, EMB)


# speedup vs baseline: 4.2041x; 1.0000x over previous
import functools

import jax
import jax.numpy as jnp
from jax import lax
from jax.experimental import pallas as pl
from jax.experimental.pallas import tpu as pltpu
from jax.experimental.pallas import tpu_sc as plsc

VOCAB = 100000
EMB = 64
BATCH = 4096
SEQ = 200

NC = 2
NS = 16
NW = NC * NS
ROWS_W = BATCH // NW
GR = 4
NCHUNK = ROWS_W // GR
NBUF = 2
NGROUPS = NCHUNK // NBUF

_mesh = plsc.VectorSubcoreMesh(core_axis_name="c", subcore_axis_name="s")


@functools.partial(
    pl.kernel,
    mesh=_mesh,
    out_type=jax.ShapeDtypeStruct((BATCH, SEQ, EMB), jnp.float32),
    compiler_params=pltpu.CompilerParams(use_tc_tiling_on_sc=False),
    scratch_types=[
        pltpu.VMEM((ROWS_W, SEQ), jnp.int32),
        pltpu.VMEM((NBUF, GR, SEQ, EMB), jnp.float32),
        pltpu.SemaphoreType.DMA,
        pltpu.SemaphoreType.DMA,
        pltpu.SemaphoreType.DMA,
        pltpu.SemaphoreType.DMA,
    ],
)
def _gather_kernel(idx_hbm, table_hbm, out_hbm, idx_v, rows_v, g0, g1, o0, o1):
    gsem = (g0, g1)
    osem = (o0, o1)
    wid = lax.axis_index("s") * NC + lax.axis_index("c")
    base = wid * ROWS_W
    pltpu.sync_copy(idx_hbm.at[pl.ds(base, ROWS_W)], idx_v)

    def gather_descs(i, b):
        return [
            pltpu.make_async_copy(
                table_hbm.at[idx_v.at[i * GR + j]], rows_v.at[b, j], gsem[b]
            )
            for j in range(GR)
        ]

    def wb_desc(i, b):
        return pltpu.make_async_copy(
            rows_v.at[b], out_hbm.at[pl.ds(base + i * GR, GR)], osem[b]
        )

    for b in range(NBUF):
        for d in gather_descs(b, b):
            d.start()

    def group(g, carry):
        i0 = g * NBUF
        for b in range(NBUF):
            for d in gather_descs(i0 + b, b):
                d.wait()
            wb_desc(i0 + b, b).start()
        for b in range(NBUF):
            wb_desc(i0 + b, b).wait()
            for d in gather_descs(i0 + NBUF + b, b):
                d.start()
        return carry

    lax.fori_loop(0, NGROUPS - 1, group, 0)

    i0 = (NGROUPS - 1) * NBUF
    for b in range(NBUF):
        for d in gather_descs(i0 + b, b):
            d.wait()
        wb_desc(i0 + b, b).start()
    for b in range(NBUF):
        wb_desc(i0 + b, b).wait()


def kernel(token_ids, table):
    return _gather_kernel(token_ids.astype(jnp.int32), table)
